# baseline (speedup 1.0000x reference)
import functools

import jax
import jax.numpy as jnp
from jax import lax
from jax.experimental import pallas as pl
from jax.experimental.pallas import tpu as pltpu
from jax.experimental.pallas import tpu_sc as plsc

_D = 64
_START = 400001
_END = 400002

_NC, _NS = 2, 16
_NW = _NC * _NS
_CHUNK = 128
_LANES = 16
_NBUF = 5
_DP = 128


def _body(n_chunks, idx_hbm, table_hbm, marker_hbm, out_hbm,
          idx_v, marker_v, rows, sem_g, sem_w):
  wid = lax.axis_index("s") * _NC + lax.axis_index("c")
  chunk0 = wid * n_chunks

  pltpu.sync_copy(idx_hbm.at[pl.ds(chunk0, n_chunks)], idx_v)
  pltpu.sync_copy(marker_hbm, marker_v)
  m0 = [marker_v[0, pl.ds(k * _LANES, _LANES)] for k in range(_D // _LANES)]
  m1 = [marker_v[1, pl.ds(k * _LANES, _LANES)] for k in range(_D // _LANES)]

  def fire_gather(b, j):
    pltpu.async_copy(table_hbm.at[idx_v.at[j]], rows[b], sem_g[b])

  def wait_gather(b, j):
    pltpu.make_async_copy(table_hbm.at[idx_v.at[j]], rows[b], sem_g[b]).wait()

  def out_dst(j):
    return out_hbm.at[pl.ds((chunk0 + j) * _CHUNK, _CHUNK)]

  def fire_write(b, j):
    pltpu.async_copy(rows[b], out_dst(j), sem_w[b])

  def wait_write(b, j):
    pltpu.make_async_copy(rows[b], out_dst(j), sem_w[b]).wait()

  def marker_check(j):
    acc = None
    for g in range(_CHUNK // _LANES):
      vg = idx_v[j, pl.ds(g * _LANES, _LANES)]
      mg = (vg == _START) | (vg == _END)
      acc = mg if acc is None else (acc | mg)
    return plsc.all_reduce_population_count(acc)[0] > 0

  def fix_markers(b, j, any_hit):
    @pl.when(any_hit)
    def _fix():
      @pl.loop(0, _CHUNK // _LANES)
      def _grp(g):
        vg = idx_v[j, pl.ds(g * _LANES, _LANES)]
        for r in range(_LANES):
          s = vg[r]
          row = g * _LANES + r

          @pl.when(s == _START)
          def _():
            for k in range(_D // _LANES):
              rows[b][row, pl.ds(k * _LANES, _LANES)] = m0[k]

          @pl.when(s == _END)
          def _():
            for k in range(_D // _LANES):
              rows[b][row, pl.ds(k * _LANES, _LANES)] = m1[k]

  for b in range(_NBUF):
    fire_gather(b, b)

  @pl.loop(0, n_chunks // _NBUF)
  def _super(js):
    for bi in range(_NBUF):
      j = js * _NBUF + bi

      bp = (bi - 1) % _NBUF
      can_prefetch = jnp.logical_and(j >= 1, j <= n_chunks - _NBUF)

      @pl.when(can_prefetch)
      def _prefetch():
        wait_write(bp, j - 1)
        fire_gather(bp, j + _NBUF - 1)

      any_hit = marker_check(j)
      wait_gather(bi, j)
      fix_markers(bi, j, any_hit)
      fire_write(bi, j)

  for b in range(_NBUF):
    wait_write(b, n_chunks - _NBUF + b)


def kernel(idxes, embeddings_weight, marker_weight):
  n_rows = idxes.size
  n_chunks = n_rows // (_NW * _CHUNK)
  assert n_rows == _NW * _CHUNK * n_chunks and n_chunks % _NBUF == 0
  idx_flat = idxes.reshape(_NW * n_chunks, _CHUNK)
  table_p = jnp.pad(embeddings_weight, ((0, 0), (0, _DP - _D)))
  marker_p = jnp.pad(marker_weight, ((0, 0), (0, _DP - _D)))

  run = pl.kernel(
      functools.partial(_body, n_chunks),
      out_type=jax.ShapeDtypeStruct((n_rows, _DP), jnp.float32),
      mesh=plsc.VectorSubcoreMesh(core_axis_name="c", subcore_axis_name="s"),
      compiler_params=pltpu.CompilerParams(
          needs_layout_passes=False, use_tc_tiling_on_sc=True),
      scratch_types=[
          pltpu.VMEM((n_chunks, _CHUNK), jnp.int32),
          pltpu.VMEM((2, _DP), jnp.float32),
          [pltpu.VMEM((_CHUNK, _DP), jnp.float32) for _ in range(_NBUF)],
          [pltpu.SemaphoreType.DMA for _ in range(_NBUF)],
          [pltpu.SemaphoreType.DMA for _ in range(_NBUF)],
      ],
  )
  out = run(idx_flat, table_p, marker_p)
  return out[:, :_D].reshape(idxes.shape + (_D,))


# speedup vs baseline: 1.0013x; 1.0013x over previous
import functools

import jax
import jax.numpy as jnp
from jax import lax
from jax.experimental import pallas as pl
from jax.experimental.pallas import tpu as pltpu
from jax.experimental.pallas import tpu_sc as plsc

_D = 64
_START = 400001
_END = 400002

_NC, _NS = 2, 16
_NW = _NC * _NS
_CHUNK = 128
_LANES = 16
_GPB = 2
_NBUF = 2
_ROWS_B = _GPB * _CHUNK
_DP = 128


def _body(n_groups, idx_hbm, table_hbm, marker_hbm, out_hbm,
          idx_v, marker_v, rows, sem_g, sem_w):
  wid = lax.axis_index("s") * _NC + lax.axis_index("c")
  n_chunks = n_groups * _GPB
  chunk0 = wid * n_chunks

  pltpu.sync_copy(idx_hbm.at[pl.ds(chunk0, n_chunks)], idx_v)
  pltpu.sync_copy(marker_hbm, marker_v)
  m0 = [marker_v[0, pl.ds(k * _LANES, _LANES)] for k in range(_D // _LANES)]
  m1 = [marker_v[1, pl.ds(k * _LANES, _LANES)] for k in range(_D // _LANES)]

  def fire_gather(b, g):
    for u in range(_GPB):
      pltpu.async_copy(table_hbm.at[idx_v.at[g * _GPB + u]],
                       rows[b].at[pl.ds(u * _CHUNK, _CHUNK)], sem_g[b])

  def wait_gather(b, g):
    for u in range(_GPB):
      pltpu.make_async_copy(table_hbm.at[idx_v.at[g * _GPB + u]],
                            rows[b].at[pl.ds(u * _CHUNK, _CHUNK)],
                            sem_g[b]).wait()

  def out_dst(g):
    return out_hbm.at[pl.ds((wid * n_groups + g) * _ROWS_B, _ROWS_B)]

  def fire_write(b, g):
    pltpu.async_copy(rows[b], out_dst(g), sem_w[b])

  def wait_write(b, g):
    pltpu.make_async_copy(rows[b], out_dst(g), sem_w[b]).wait()

  def marker_check(g):
    acc = None
    for u in range(_GPB):
      for q in range(_CHUNK // _LANES):
        vg = idx_v[g * _GPB + u, pl.ds(q * _LANES, _LANES)]
        mg = (vg == _START) | (vg == _END)
        acc = mg if acc is None else (acc | mg)
    return plsc.all_reduce_population_count(acc)[0] > 0

  def fix_markers(b, g, any_hit):
    @pl.when(any_hit)
    def _fix():
      for u in range(_GPB):
        @pl.loop(0, _CHUNK // _LANES)
        def _grp(q):
          vg = idx_v[g * _GPB + u, pl.ds(q * _LANES, _LANES)]
          for r in range(_LANES):
            s = vg[r]
            row = u * _CHUNK + q * _LANES + r

            @pl.when(s == _START)
            def _():
              for k in range(_D // _LANES):
                rows[b][row, pl.ds(k * _LANES, _LANES)] = m0[k]

            @pl.when(s == _END)
            def _():
              for k in range(_D // _LANES):
                rows[b][row, pl.ds(k * _LANES, _LANES)] = m1[k]

  for b in range(_NBUF):
    fire_gather(b, b)

  @pl.loop(0, n_groups // _NBUF)
  def _super(js):
    for bi in range(_NBUF):
      g = js * _NBUF + bi

      bp = (bi - 1) % _NBUF
      can_prefetch = jnp.logical_and(g >= 1, g <= n_groups - _NBUF)

      @pl.when(can_prefetch)
      def _prefetch():
        wait_write(bp, g - 1)
        fire_gather(bp, g + _NBUF - 1)

      any_hit = marker_check(g)
      wait_gather(bi, g)
      fix_markers(bi, g, any_hit)
      fire_write(bi, g)

  for b in range(_NBUF):
    wait_write(b, n_groups - _NBUF + b)


def kernel(idxes, embeddings_weight, marker_weight):
  n_rows = idxes.size
  n_chunks = n_rows // (_NW * _CHUNK)
  n_groups = n_chunks // _GPB
  assert n_rows == _NW * _CHUNK * n_chunks and n_groups % _NBUF == 0
  idx_flat = idxes.reshape(_NW * n_chunks, _CHUNK)
  table_p = jnp.pad(embeddings_weight, ((0, 0), (0, _DP - _D)))
  marker_p = jnp.pad(marker_weight, ((0, 0), (0, _DP - _D)))

  run = pl.kernel(
      functools.partial(_body, n_groups),
      out_type=jax.ShapeDtypeStruct((n_rows, _DP), jnp.float32),
      mesh=plsc.VectorSubcoreMesh(core_axis_name="c", subcore_axis_name="s"),
      compiler_params=pltpu.CompilerParams(
          needs_layout_passes=False, use_tc_tiling_on_sc=True),
      scratch_types=[
          pltpu.VMEM((n_chunks, _CHUNK), jnp.int32),
          pltpu.VMEM((2, _DP), jnp.float32),
          [pltpu.VMEM((_ROWS_B, _DP), jnp.float32) for _ in range(_NBUF)],
          [pltpu.SemaphoreType.DMA for _ in range(_NBUF)],
          [pltpu.SemaphoreType.DMA for _ in range(_NBUF)],
      ],
  )
  out = run(idx_flat, table_p, marker_p)
  return out[:, :_D].reshape(idxes.shape + (_D,))


# speedup vs baseline: 1.0907x; 1.0893x over previous
import functools

import jax
import jax.numpy as jnp
from jax import lax
from jax.experimental import pallas as pl
from jax.experimental.pallas import tpu as pltpu
from jax.experimental.pallas import tpu_sc as plsc

_D = 64
_START = 400001
_END = 400002

_NC, _NS = 2, 16
_NW = _NC * _NS
_CHUNK = 128
_LANES = 16
_GPB = 2
_NBUF = 2
_ROWS_B = _GPB * _CHUNK
_DP = 128


def _body(n_groups, idx_hbm, table_hbm, marker_hbm, out_hbm,
          idx_v, marker_v, rows, sem_g, sem_w):
  wid = lax.axis_index("s") * _NC + lax.axis_index("c")
  n_chunks = n_groups * _GPB
  chunk0 = wid * n_chunks

  pltpu.sync_copy(idx_hbm.at[pl.ds(chunk0, n_chunks)], idx_v)
  pltpu.sync_copy(marker_hbm, marker_v)
  m0 = [marker_v[0, pl.ds(k * _LANES, _LANES)] for k in range(_D // _LANES)]
  m1 = [marker_v[1, pl.ds(k * _LANES, _LANES)] for k in range(_D // _LANES)]

  def fire_gather(b, g):
    for u in range(_GPB):
      pltpu.async_copy(table_hbm.at[idx_v.at[g * _GPB + u]],
                       rows[b].at[pl.ds(u * _CHUNK, _CHUNK)], sem_g[b])

  def wait_gather(b, g):
    for u in range(_GPB):
      pltpu.make_async_copy(table_hbm.at[idx_v.at[g * _GPB + u]],
                            rows[b].at[pl.ds(u * _CHUNK, _CHUNK)],
                            sem_g[b]).wait()

  def out_dst(g):
    return out_hbm.at[pl.ds((wid * n_groups + g) * _ROWS_B, _ROWS_B)]

  def fire_write(b, g):
    pltpu.async_copy(rows[b], out_dst(g), sem_w[b])

  def wait_write(b, g):
    pltpu.make_async_copy(rows[b], out_dst(g), sem_w[b]).wait()

  def marker_check(g):
    acc = None
    for u in range(_GPB):
      for q in range(_CHUNK // _LANES):
        vg = idx_v[g * _GPB + u, pl.ds(q * _LANES, _LANES)]
        mg = (vg == _START) | (vg == _END)
        acc = mg if acc is None else (acc | mg)
    return plsc.all_reduce_population_count(acc)[0] > 0

  def fix_markers(b, g, any_hit):
    @pl.when(any_hit)
    def _fix():
      for u in range(_GPB):
        @pl.loop(0, _CHUNK // _LANES)
        def _grp(q):
          vg = idx_v[g * _GPB + u, pl.ds(q * _LANES, _LANES)]
          for r in range(_LANES):
            s = vg[r]
            row = u * _CHUNK + q * _LANES + r

            @pl.when(s == _START)
            def _():
              for k in range(_D // _LANES):
                rows[b][row, pl.ds(k * _LANES, _LANES)] = m0[k]

            @pl.when(s == _END)
            def _():
              for k in range(_D // _LANES):
                rows[b][row, pl.ds(k * _LANES, _LANES)] = m1[k]

  for b in range(_NBUF):
    fire_gather(b, b)

  @pl.loop(0, n_groups // _NBUF)
  def _super(js):
    for bi in range(_NBUF):
      g = js * _NBUF + bi

      bp = (bi - 1) % _NBUF
      can_prefetch = jnp.logical_and(g >= 1, g <= n_groups - _NBUF)

      @pl.when(can_prefetch)
      def _prefetch():
        wait_write(bp, g - 1)
        fire_gather(bp, g + _NBUF - 1)

      any_hit = marker_check(g)
      wait_gather(bi, g)
      fix_markers(bi, g, any_hit)
      fire_write(bi, g)

  for b in range(_NBUF):
    wait_write(b, n_groups - _NBUF + b)


_VB = 4096


def _pad_body(in_ref, out_ref):
  out_ref[:, 0:_D] = in_ref[...].T
  out_ref[:, _D:] = jnp.zeros((_VB, _DP - _D), jnp.float32)


def _transpose_pad(table_t):
  v = table_t.shape[1]
  grid = (v + _VB - 1) // _VB
  return pl.pallas_call(
      _pad_body,
      grid=(grid,),
      in_specs=[pl.BlockSpec((_D, _VB), lambda i: (0, i))],
      out_specs=pl.BlockSpec((_VB, _DP), lambda i: (i, 0)),
      out_shape=jax.ShapeDtypeStruct((v, _DP), jnp.float32),
  )(table_t)


def kernel(idxes, embeddings_weight, marker_weight):
  n_rows = idxes.size
  n_chunks = n_rows // (_NW * _CHUNK)
  n_groups = n_chunks // _GPB
  assert n_rows == _NW * _CHUNK * n_chunks and n_groups % _NBUF == 0
  idx_flat = idxes.reshape(_NW * n_chunks, _CHUNK)
  table_p = _transpose_pad(embeddings_weight.T)
  marker_p = jnp.pad(marker_weight, ((0, 0), (0, _DP - _D)))

  run = pl.kernel(
      functools.partial(_body, n_groups),
      out_type=jax.ShapeDtypeStruct((n_rows, _DP), jnp.float32),
      mesh=plsc.VectorSubcoreMesh(core_axis_name="c", subcore_axis_name="s"),
      compiler_params=pltpu.CompilerParams(
          needs_layout_passes=False, use_tc_tiling_on_sc=True),
      scratch_types=[
          pltpu.VMEM((n_chunks, _CHUNK), jnp.int32),
          pltpu.VMEM((2, _DP), jnp.float32),
          [pltpu.VMEM((_ROWS_B, _DP), jnp.float32) for _ in range(_NBUF)],
          [pltpu.SemaphoreType.DMA for _ in range(_NBUF)],
          [pltpu.SemaphoreType.DMA for _ in range(_NBUF)],
      ],
  )
  out = run(idx_flat, table_p, marker_p)
  return out[:, :_D].reshape(idxes.shape + (_D,))


# speedup vs baseline: 1.0908x; 1.0001x over previous
import functools

import jax
import jax.numpy as jnp
from jax import lax
from jax.experimental import pallas as pl
from jax.experimental.pallas import tpu as pltpu
from jax.experimental.pallas import tpu_sc as plsc

_D = 64
_START = 400001
_END = 400002

_NC, _NS = 2, 16
_NW = _NC * _NS
_CHUNK = 128
_LANES = 16
_GPB = 2
_NBUF = 2
_ROWS_B = _GPB * _CHUNK
_DP = 128


def _body(n_groups, idx_hbm, table_hbm, marker_hbm, out_hbm,
          idx_v, marker_v, rows, sem_g, sem_w):
  wid = lax.axis_index("s") * _NC + lax.axis_index("c")
  n_chunks = n_groups * _GPB
  chunk0 = wid * n_chunks

  pltpu.sync_copy(idx_hbm.at[pl.ds(chunk0, n_chunks)], idx_v)
  pltpu.sync_copy(marker_hbm, marker_v)
  m0 = [marker_v[0, pl.ds(k * _LANES, _LANES)] for k in range(_D // _LANES)]
  m1 = [marker_v[1, pl.ds(k * _LANES, _LANES)] for k in range(_D // _LANES)]

  def fire_gather(b, g):
    for u in range(_GPB):
      pltpu.async_copy(table_hbm.at[idx_v.at[g * _GPB + u]],
                       rows[b].at[pl.ds(u * _CHUNK, _CHUNK)], sem_g[b])

  def wait_gather(b, g):
    for u in range(_GPB):
      pltpu.make_async_copy(table_hbm.at[idx_v.at[g * _GPB + u]],
                            rows[b].at[pl.ds(u * _CHUNK, _CHUNK)],
                            sem_g[b]).wait()

  def out_dst(g):
    return out_hbm.at[pl.ds((wid * n_groups + g) * _ROWS_B, _ROWS_B)]

  def fire_write(b, g):
    pltpu.async_copy(rows[b], out_dst(g), sem_w[b])

  def wait_write(b, g):
    pltpu.make_async_copy(rows[b], out_dst(g), sem_w[b]).wait()

  def marker_check(g):
    acc = None
    for u in range(_GPB):
      for q in range(_CHUNK // _LANES):
        vg = idx_v[g * _GPB + u, pl.ds(q * _LANES, _LANES)]
        mg = (vg == _START) | (vg == _END)
        acc = mg if acc is None else (acc | mg)
    return plsc.all_reduce_population_count(acc)[0] > 0

  def fix_markers(b, g, any_hit):
    @pl.when(any_hit)
    def _fix():
      for u in range(_GPB):
        @pl.loop(0, _CHUNK // _LANES)
        def _grp(q):
          vg = idx_v[g * _GPB + u, pl.ds(q * _LANES, _LANES)]
          for r in range(_LANES):
            s = vg[r]
            row = u * _CHUNK + q * _LANES + r

            @pl.when(s == _START)
            def _():
              for k in range(_D // _LANES):
                rows[b][row, pl.ds(k * _LANES, _LANES)] = m0[k]

            @pl.when(s == _END)
            def _():
              for k in range(_D // _LANES):
                rows[b][row, pl.ds(k * _LANES, _LANES)] = m1[k]

  for b in range(_NBUF):
    fire_gather(b, b)

  @pl.loop(0, n_groups // _NBUF)
  def _super(js):
    for bi in range(_NBUF):
      g = js * _NBUF + bi

      bp = (bi - 1) % _NBUF
      can_prefetch = jnp.logical_and(g >= 1, g <= n_groups - _NBUF)

      @pl.when(can_prefetch)
      def _prefetch():
        wait_write(bp, g - 1)
        fire_gather(bp, g + _NBUF - 1)

      any_hit = marker_check(g)
      wait_gather(bi, g)
      fix_markers(bi, g, any_hit)
      fire_write(bi, g)

  for b in range(_NBUF):
    wait_write(b, n_groups - _NBUF + b)


_VB = 4096


def _pad_body(in_ref, out_ref):
  out_ref[:, 0:_D] = in_ref[...].T


def _transpose_pad(table_t):
  v = table_t.shape[1]
  grid = (v + _VB - 1) // _VB
  return pl.pallas_call(
      _pad_body,
      grid=(grid,),
      in_specs=[pl.BlockSpec((_D, _VB), lambda i: (0, i))],
      out_specs=pl.BlockSpec((_VB, _DP), lambda i: (i, 0)),
      out_shape=jax.ShapeDtypeStruct((v, _DP), jnp.float32),
  )(table_t)


def kernel(idxes, embeddings_weight, marker_weight):
  n_rows = idxes.size
  n_chunks = n_rows // (_NW * _CHUNK)
  n_groups = n_chunks // _GPB
  assert n_rows == _NW * _CHUNK * n_chunks and n_groups % _NBUF == 0
  idx_flat = idxes.reshape(_NW * n_chunks, _CHUNK)
  table_p = _transpose_pad(embeddings_weight.T)
  marker_p = jnp.pad(marker_weight, ((0, 0), (0, _DP - _D)))

  run = pl.kernel(
      functools.partial(_body, n_groups),
      out_type=jax.ShapeDtypeStruct((n_rows, _DP), jnp.float32),
      mesh=plsc.VectorSubcoreMesh(core_axis_name="c", subcore_axis_name="s"),
      compiler_params=pltpu.CompilerParams(
          needs_layout_passes=False, use_tc_tiling_on_sc=True),
      scratch_types=[
          pltpu.VMEM((n_chunks, _CHUNK), jnp.int32),
          pltpu.VMEM((2, _DP), jnp.float32),
          [pltpu.VMEM((_ROWS_B, _DP), jnp.float32) for _ in range(_NBUF)],
          [pltpu.SemaphoreType.DMA for _ in range(_NBUF)],
          [pltpu.SemaphoreType.DMA for _ in range(_NBUF)],
      ],
  )
  out = run(idx_flat, table_p, marker_p)
  return out[:, :_D].reshape(idxes.shape + (_D,))
